# baseline (speedup 1.0000x reference)
import jax
import jax.numpy as jnp
from jax import lax
from jax.experimental import pallas as pl
from jax.experimental.pallas import tpu as pltpu
from jax.experimental.pallas import tpu_sc as plsc

_OFFSET = 500000
_B = 16384
_NC, _NS, _L = 2, 16, 16
_NW = _NC * _NS
_BPW = _B // _NW
_SIZES = (64, 128, 128, 128, 64)
_OFFS = tuple(sum(_SIZES[:i]) for i in range(len(_SIZES)))
_NCHUNK = len(_SIZES)


def _body(x_hbm, tab_hbm, bias_hbm, out_hbm, *scratch):
    bufs = [scratch[6 * c:6 * c + 6] for c in range(_NCHUNK)]
    bias_v = scratch[6 * _NCHUNK]
    sems = scratch[6 * _NCHUNK + 1:6 * _NCHUNK + 1 + _NCHUNK]
    semo = scratch[6 * _NCHUNK + 1 + _NCHUNK]

    wid = lax.axis_index("s") * _NC + lax.axis_index("c")
    base = wid * _BPW

    hx = []
    for c in range(_NCHUNK):
        hx.append(pltpu.async_copy(
            x_hbm.at[pl.ds(2 * (base + _OFFS[c]), 2 * _SIZES[c])],
            bufs[c][0], sems[c]))
    hb = pltpu.async_copy(bias_hbm, bias_v, semo)

    lanes = lax.iota(jnp.int32, _L)
    evens = (lanes * 2) & (_L - 1)
    odds = evens + 1
    lo_half = lanes < 8

    gathers = []
    for c in range(_NCHUNK):
        x_v, iu_v, im_v, ru_v, rm_v, y_v = bufs[c]
        hx[c].wait()
        for v in range(_SIZES[c] // _L):
            a = x_v[pl.ds(v * 2 * _L, _L)]
            b = x_v[pl.ds(v * 2 * _L + _L, _L)]
            u = jnp.where(lo_half,
                          a.at[evens].get(mode="promise_in_bounds"),
                          b.at[evens].get(mode="promise_in_bounds"))
            m = jnp.where(lo_half,
                          a.at[odds].get(mode="promise_in_bounds"),
                          b.at[odds].get(mode="promise_in_bounds"))
            iu_v[pl.ds(v * _L, _L)] = u
            im_v[pl.ds(v * _L, _L)] = m + _OFFSET
        gathers.append(pltpu.async_copy(tab_hbm.at[iu_v], ru_v, sems[c]))
        gathers.append(pltpu.async_copy(tab_hbm.at[im_v], rm_v, sems[c]))

    hb.wait()
    bias_vec = bias_v[...]

    ho = []
    for c in range(_NCHUNK):
        x_v, iu_v, im_v, ru_v, rm_v, y_v = bufs[c]
        gathers[2 * c].wait()
        gathers[2 * c + 1].wait()
        for v in range(_SIZES[c] // _L):
            s = pl.ds(v * _L, _L)
            y_v[s] = ru_v[s] + rm_v[s] + bias_vec
        ho.append(pltpu.async_copy(
            y_v, out_hbm.at[pl.ds(base + _OFFS[c], _SIZES[c])], semo))
    for h in ho:
        h.wait()


def kernel(x, fc_weight, bias):
    mesh = plsc.VectorSubcoreMesh(core_axis_name="c", subcore_axis_name="s")
    scratch = []
    for sz in _SIZES:
        scratch += [
            pltpu.VMEM((2 * sz,), jnp.int32),
            pltpu.VMEM((sz,), jnp.int32),
            pltpu.VMEM((sz,), jnp.int32),
            pltpu.VMEM((sz,), jnp.float32),
            pltpu.VMEM((sz,), jnp.float32),
            pltpu.VMEM((sz,), jnp.float32),
        ]
    scratch.append(pltpu.VMEM((_L,), jnp.float32))
    scratch += [pltpu.SemaphoreType.DMA] * (_NCHUNK + 1)
    k = pl.kernel(
        _body,
        mesh=mesh,
        out_type=jax.ShapeDtypeStruct((_B,), jnp.float32),
        scratch_types=scratch,
    )
    x_flat = x.reshape(-1).astype(jnp.int32)
    tab = fc_weight.reshape(-1)
    bias16 = jnp.broadcast_to(bias.astype(jnp.float32), (_L,))
    y = k(x_flat, tab, bias16)
    return y.reshape(_B, 1)


# speedup vs baseline: 1.0003x; 1.0003x over previous
import jax
import jax.numpy as jnp
from jax import lax
from jax.experimental import pallas as pl
from jax.experimental.pallas import tpu as pltpu
from jax.experimental.pallas import tpu_sc as plsc

_OFFSET = 500000
_B = 16384
_NC, _NS, _L = 2, 16, 16
_NW = _NC * _NS
_BPW = _B // _NW
_NCHUNK = 4
_CROWS = _BPW // _NCHUNK
_CVEC = _CROWS // _L


def _body(x_hbm, tab_hbm, bias_hbm, out_hbm,
          x_v, iu_v, im_v, ru_v, rm_v, y_v, bias_v,
          sem0, sem1, sem2, sem3, semo):
    wid = lax.axis_index("s") * _NC + lax.axis_index("c")
    base = wid * _BPW
    sems = [sem0, sem1, sem2, sem3]

    hx = []
    for c in range(_NCHUNK):
        hx.append(pltpu.async_copy(
            x_hbm.at[pl.ds(2 * (base + c * _CROWS), 2 * _CROWS)],
            x_v.at[c], sems[c]))
    hb = pltpu.async_copy(bias_hbm, bias_v, semo)

    lanes = lax.iota(jnp.int32, _L)
    evens = (lanes * 2) & (_L - 1)
    odds = evens + 1
    lo_half = lanes < 8

    gathers = []
    for c in range(_NCHUNK):
        hx[c].wait()
        for v in range(_CVEC):
            a = x_v[c, pl.ds(v * 2 * _L, _L)]
            b = x_v[c, pl.ds(v * 2 * _L + _L, _L)]
            u = jnp.where(lo_half,
                          a.at[evens].get(mode="promise_in_bounds"),
                          b.at[evens].get(mode="promise_in_bounds"))
            m = jnp.where(lo_half,
                          a.at[odds].get(mode="promise_in_bounds"),
                          b.at[odds].get(mode="promise_in_bounds"))
            iu_v[c, pl.ds(v * _L, _L)] = u
            im_v[c, pl.ds(v * _L, _L)] = m + _OFFSET
        gathers.append(pltpu.async_copy(
            tab_hbm.at[iu_v.at[c]], ru_v.at[c], sems[c]))
        gathers.append(pltpu.async_copy(
            tab_hbm.at[im_v.at[c]], rm_v.at[c], sems[c]))

    hb.wait()
    bias_vec = bias_v[...]

    ho = []
    for c in range(_NCHUNK):
        gathers[2 * c].wait()
        gathers[2 * c + 1].wait()
        for v in range(_CVEC):
            s = pl.ds(v * _L, _L)
            y_v[c, s] = ru_v[c, s] + rm_v[c, s] + bias_vec
        ho.append(pltpu.async_copy(
            y_v.at[c], out_hbm.at[pl.ds(base + c * _CROWS, _CROWS)], semo))
    for h in ho:
        h.wait()


def kernel(x, fc_weight, bias):
    mesh = plsc.VectorSubcoreMesh(core_axis_name="c", subcore_axis_name="s")
    k = pl.kernel(
        _body,
        mesh=mesh,
        out_type=jax.ShapeDtypeStruct((_B,), jnp.float32),
        scratch_types=[
            pltpu.VMEM((_NCHUNK, 2 * _CROWS), jnp.int32),
            pltpu.VMEM((_NCHUNK, _CROWS), jnp.int32),
            pltpu.VMEM((_NCHUNK, _CROWS), jnp.int32),
            pltpu.VMEM((_NCHUNK, _CROWS), jnp.float32),
            pltpu.VMEM((_NCHUNK, _CROWS), jnp.float32),
            pltpu.VMEM((_NCHUNK, _CROWS), jnp.float32),
            pltpu.VMEM((_L,), jnp.float32),
            pltpu.SemaphoreType.DMA,
            pltpu.SemaphoreType.DMA,
            pltpu.SemaphoreType.DMA,
            pltpu.SemaphoreType.DMA,
            pltpu.SemaphoreType.DMA,
        ],
    )
    x_flat = x.reshape(-1).astype(jnp.int32)
    tab = fc_weight.reshape(-1)
    bias16 = jnp.broadcast_to(bias.astype(jnp.float32), (_L,))
    y = k(x_flat, tab, bias16)
    return y.reshape(_B, 1)


# speedup vs baseline: 1.0008x; 1.0006x over previous
import jax
import jax.numpy as jnp
from jax import lax
from jax.experimental import pallas as pl
from jax.experimental.pallas import tpu as pltpu
from jax.experimental.pallas import tpu_sc as plsc

_OFFSET = 500000
_B = 16384
_NC, _NS, _L = 2, 16, 16
_NW = _NC * _NS
_BPW = _B // _NW
_NCHUNK = 4
_CROWS = _BPW // _NCHUNK
_CVEC = _CROWS // _L


def _body(x_hbm, tab_hbm, bias_hbm, out_hbm, *scratch):
    bufs = [scratch[4 * c:4 * c + 4] for c in range(_NCHUNK)]
    bias_v = scratch[4 * _NCHUNK]
    sems = scratch[4 * _NCHUNK + 1:4 * _NCHUNK + 1 + _NCHUNK]
    semo = scratch[4 * _NCHUNK + 1 + _NCHUNK]

    wid = lax.axis_index("s") * _NC + lax.axis_index("c")
    base = wid * _BPW

    hx = []
    for c in range(_NCHUNK):
        hx.append(pltpu.async_copy(
            x_hbm.at[pl.ds(2 * (base + c * _CROWS), 2 * _CROWS)],
            bufs[c][0], sems[c]))
    hb = pltpu.async_copy(bias_hbm, bias_v, semo)

    lanes = lax.iota(jnp.int32, _L)
    evens = (lanes * 2) & (_L - 1)
    odds = evens + 1
    lo_half = lanes < 8

    gathers = []
    for c in range(_NCHUNK):
        x_v, ic_v, r_v, y_v = bufs[c]
        hx[c].wait()
        for v in range(_CVEC):
            a = x_v[pl.ds(v * 2 * _L, _L)]
            b = x_v[pl.ds(v * 2 * _L + _L, _L)]
            u = jnp.where(lo_half,
                          a.at[evens].get(mode="promise_in_bounds"),
                          b.at[evens].get(mode="promise_in_bounds"))
            m = jnp.where(lo_half,
                          a.at[odds].get(mode="promise_in_bounds"),
                          b.at[odds].get(mode="promise_in_bounds"))
            ic_v[pl.ds(v * _L, _L)] = u
            ic_v[pl.ds(_CROWS + v * _L, _L)] = m + _OFFSET
        gathers.append(pltpu.async_copy(tab_hbm.at[ic_v], r_v, sems[c]))

    hb.wait()
    bias_vec = bias_v[...]

    ho = []
    for c in range(_NCHUNK):
        x_v, ic_v, r_v, y_v = bufs[c]
        gathers[c].wait()
        for v in range(_CVEC):
            s = pl.ds(v * _L, _L)
            y_v[s] = (r_v[s] + r_v[pl.ds(_CROWS + v * _L, _L)] + bias_vec)
        ho.append(pltpu.async_copy(
            y_v, out_hbm.at[pl.ds(base + c * _CROWS, _CROWS)], semo))
    for h in ho:
        h.wait()


def kernel(x, fc_weight, bias):
    mesh = plsc.VectorSubcoreMesh(core_axis_name="c", subcore_axis_name="s")
    scratch = []
    for _ in range(_NCHUNK):
        scratch += [
            pltpu.VMEM((2 * _CROWS,), jnp.int32),
            pltpu.VMEM((2 * _CROWS,), jnp.int32),
            pltpu.VMEM((2 * _CROWS,), jnp.float32),
            pltpu.VMEM((_CROWS,), jnp.float32),
        ]
    scratch.append(pltpu.VMEM((_L,), jnp.float32))
    scratch += [pltpu.SemaphoreType.DMA] * (_NCHUNK + 1)
    k = pl.kernel(
        _body,
        mesh=mesh,
        out_type=jax.ShapeDtypeStruct((_B,), jnp.float32),
        scratch_types=scratch,
    )
    x_flat = x.reshape(-1).astype(jnp.int32)
    tab = fc_weight.reshape(-1)
    bias16 = jnp.broadcast_to(bias.astype(jnp.float32), (_L,))
    y = k(x_flat, tab, bias16)
    return y.reshape(_B, 1)
